# Initial kernel scaffold; baseline (speedup 1.0000x reference)
import jax
import jax.numpy as jnp
from jax.experimental import pallas as pl


def kernel(x, mu, logD):
    raise NotImplementedError("write your pallas kernel here")


# speedup vs baseline: 1.0037x; 1.0037x over previous
import functools

import jax
import jax.numpy as jnp
from jax.experimental import pallas as pl

_DIM = 2048
_KEEP_RANK = _DIM // 2
_EPS = 1e-08
_CHUNK = 256


def _mask_body(mu_row_ref, logD_row_ref, mu_col_ref, logD_col_ref, mask_ref):
    la_row = logD_row_ref[...] - jnp.log(mu_row_ref[...] ** 2 + _EPS)
    la_col = logD_col_ref[...] - jnp.log(mu_col_ref[...] ** 2 + _EPS)
    j_idx = jax.lax.broadcasted_iota(jnp.int32, (_CHUNK, _DIM), 1)
    counts = jnp.zeros((1, _DIM), dtype=jnp.int32)
    for k in range(_DIM // _CHUNK):
        la_i = la_col[k * _CHUNK:(k + 1) * _CHUNK, :]
        i_idx = k * _CHUNK + jax.lax.broadcasted_iota(
            jnp.int32, (_CHUNK, _DIM), 0)
        less = la_i < la_row
        eq_before = (la_i == la_row) & (i_idx < j_idx)
        counts = counts + jnp.sum(
            (less | eq_before).astype(jnp.int32), axis=0, keepdims=True)
    mask_ref[...] = (counts >= _KEEP_RANK).astype(jnp.float32)


def _mul_body(x_ref, mask_ref, o_ref):
    o_ref[...] = x_ref[...] * mask_ref[...]


@functools.partial(jax.jit, static_argnames=("block_rows",))
def _run(x, mu, logD, block_rows=1024):
    mu_row = mu.reshape(1, _DIM)
    logD_row = logD.reshape(1, _DIM)
    mu_col = mu.reshape(_DIM, 1)
    logD_col = logD.reshape(_DIM, 1)

    mask = pl.pallas_call(
        _mask_body,
        out_shape=jax.ShapeDtypeStruct((1, _DIM), jnp.float32),
    )(mu_row, logD_row, mu_col, logD_col)

    rows = x.shape[0] * x.shape[1]
    x2d = x.reshape(rows, _DIM)
    y2d = pl.pallas_call(
        _mul_body,
        grid=(rows // block_rows,),
        in_specs=[
            pl.BlockSpec((block_rows, _DIM), lambda i: (i, 0)),
            pl.BlockSpec((1, _DIM), lambda i: (0, 0)),
        ],
        out_specs=pl.BlockSpec((block_rows, _DIM), lambda i: (i, 0)),
        out_shape=jax.ShapeDtypeStruct((rows, _DIM), jnp.float32),
    )(x2d, mask)
    return y2d.reshape(x.shape)


def kernel(x, mu, logD):
    return _run(x, mu, logD)


# speedup vs baseline: 1.0148x; 1.0111x over previous
import functools

import jax
import jax.numpy as jnp
from jax.experimental import pallas as pl
from jax.experimental.pallas import tpu as pltpu

_DIM = 2048
_KEEP_RANK = _DIM // 2
_EPS = 1e-08
_CHUNK = 256


def _body(mu_row_ref, logD_row_ref, mu_col_ref, logD_col_ref, x_ref, o_ref,
          mask_ref):
    @pl.when(pl.program_id(0) == 0)
    def _compute_mask():
        la_row = logD_row_ref[...] - jnp.log(mu_row_ref[...] ** 2 + _EPS)
        la_col = logD_col_ref[...] - jnp.log(mu_col_ref[...] ** 2 + _EPS)
        j_idx = jax.lax.broadcasted_iota(jnp.int32, (_CHUNK, _DIM), 1)
        counts = jnp.zeros((1, _DIM), dtype=jnp.int32)
        for k in range(_DIM // _CHUNK):
            la_i = la_col[k * _CHUNK:(k + 1) * _CHUNK, :]
            i_idx = k * _CHUNK + jax.lax.broadcasted_iota(
                jnp.int32, (_CHUNK, _DIM), 0)
            less = la_i < la_row
            eq_before = (la_i == la_row) & (i_idx < j_idx)
            counts = counts + jnp.sum(
                (less | eq_before).astype(jnp.int32), axis=0, keepdims=True)
        mask_ref[...] = (counts >= _KEEP_RANK).astype(jnp.float32)

    o_ref[...] = x_ref[...] * mask_ref[...]


@functools.partial(jax.jit, static_argnames=("block_rows",))
def _run(x, mu, logD, block_rows=1024):
    mu_row = mu.reshape(1, _DIM)
    logD_row = logD.reshape(1, _DIM)
    mu_col = mu.reshape(_DIM, 1)
    logD_col = logD.reshape(_DIM, 1)

    rows = x.shape[0] * x.shape[1]
    x2d = x.reshape(rows, _DIM)
    y2d = pl.pallas_call(
        _body,
        grid=(rows // block_rows,),
        in_specs=[
            pl.BlockSpec((1, _DIM), lambda i: (0, 0)),
            pl.BlockSpec((1, _DIM), lambda i: (0, 0)),
            pl.BlockSpec((_DIM, 1), lambda i: (0, 0)),
            pl.BlockSpec((_DIM, 1), lambda i: (0, 0)),
            pl.BlockSpec((block_rows, _DIM), lambda i: (i, 0)),
        ],
        out_specs=pl.BlockSpec((block_rows, _DIM), lambda i: (i, 0)),
        out_shape=jax.ShapeDtypeStruct((rows, _DIM), jnp.float32),
        scratch_shapes=[pltpu.VMEM((1, _DIM), jnp.float32)],
        compiler_params=pltpu.CompilerParams(
            dimension_semantics=("arbitrary",)),
    )(mu_row, logD_row, mu_col, logD_col, x2d)
    return y2d.reshape(x.shape)


def kernel(x, mu, logD):
    return _run(x, mu, logD)
